# baseline (speedup 1.0000x reference)
import functools

import jax
import jax.numpy as jnp
from jax import lax
from jax.experimental import pallas as pl
from jax.experimental.pallas import tpu as pltpu
from jax.experimental.pallas import tpu_sc as plsc

_N = 10000
_D = 128
_NPAD = 10240
_NC = 2
_NS = 16
_NW = _NC * _NS
_RPT = _NPAD // _NS
_B = 128
_DEGW = 16
_ZR = 32


@functools.cache
def _make_sc_pass(kch, compute_deg=False):
    mesh = plsc.VectorSubcoreMesh(core_axis_name="c", subcore_axis_name="s")
    out_type = [jax.ShapeDtypeStruct((3, _NC, _NPAD, _D), jnp.float32)]
    scratch = [
        pltpu.VMEM((kch, _B), jnp.int32),
        pltpu.VMEM((kch, _B), jnp.int32),
        pltpu.VMEM((2, _B, _D), jnp.float32),
        pltpu.VMEM((_ZR, _D), jnp.float32),
        pltpu.VMEM_SHARED((_NPAD, _D), jnp.float32),
        pltpu.SemaphoreType.DMA,
        pltpu.SemaphoreType.DMA,
        pltpu.SemaphoreType.DMA,
    ]
    if compute_deg:
        out_type.append(jax.ShapeDtypeStruct((3 * _NC * _NPAD,), jnp.float32))
        scratch += [
            pltpu.VMEM((_B,), jnp.float32),
            pltpu.VMEM((_RPT,), jnp.float32),
            pltpu.VMEM_SHARED((_NPAD,), jnp.float32),
        ]

    @functools.partial(pl.kernel, mesh=mesh,
                       out_type=tuple(out_type) if compute_deg else out_type[0],
                       scratch_types=tuple(scratch))
    def body(*refs):
        if compute_deg:
            (x_hbm, srcs_hbm, dsts_hbm, zrow_hbm, ones_hbm, zdeg_hbm,
             acc_out, deg_out,
             src_v, dst_v, rows_v, zac_v, acc_sh, sem0, sem1, semw,
             ones_v, zdg_v, deg_sh) = refs
        else:
            (x_hbm, srcs_hbm, dsts_hbm, zrow_hbm,
             acc_out,
             src_v, dst_v, rows_v, zac_v, acc_sh, sem0, sem1, semw) = refs
        c = lax.axis_index("c")
        s = lax.axis_index("s")
        wid = s * _NC + c
        base = s * _RPT
        npairs = kch // 2

        pltpu.sync_copy(zrow_hbm, zac_v)
        if compute_deg:
            pltpu.sync_copy(ones_hbm, ones_v)
            pltpu.sync_copy(zdeg_hbm, zdg_v)

        def gather(j, buf, sem):
            for q in range(4):
                pltpu.async_copy(x_hbm.at[src_v.at[j, pl.ds(q * 32, 32)]],
                                 rows_v.at[buf, pl.ds(q * 32, 32)], sem)

        def gwait(j, buf, sem):
            for q in range(4):
                pltpu.make_async_copy(x_hbm.at[src_v.at[j, pl.ds(q * 32, 32)]],
                                      rows_v.at[buf, pl.ds(q * 32, 32)],
                                      sem).wait()

        def scatter(j, buf):
            pltpu.sync_copy(rows_v.at[buf], acc_sh.at[dst_v.at[j]], add=True)

        def wb_acc(r):
            return pltpu.make_async_copy(acc_sh.at[pl.ds(base, _RPT)],
                                         acc_out.at[r, c, pl.ds(base, _RPT)],
                                         semw)

        def wb_deg(r):
            dst = deg_out.at[pl.ds((r * _NC + c) * _NPAD + base, _RPT)]
            return pltpu.make_async_copy(deg_sh.at[pl.ds(base, _RPT)], dst,
                                         semw)

        for r in range(3):
            pltpu.sync_copy(srcs_hbm.at[r, wid], src_v)
            pltpu.sync_copy(dsts_hbm.at[r, wid], dst_v)
            gather(0, 0, sem0)
            gather(1, 1, sem1)
            if r:
                wb_acc(r - 1).wait()
                if compute_deg:
                    wb_deg(r - 1).wait()
            for k in range(_RPT // _ZR):
                pltpu.sync_copy(zac_v, acc_sh.at[pl.ds(base + k * _ZR, _ZR)])
            if compute_deg:
                pltpu.sync_copy(zdg_v, deg_sh.at[pl.ds(base, _RPT)])
            plsc.subcore_barrier()

            def pair_body(p, carry):
                j = p * 2
                gwait(j, 0, sem0)
                scatter(j, 0)

                @pl.when(j + 2 < kch)
                def _():
                    gather(j + 2, 0, sem0)

                gwait(j + 1, 1, sem1)
                scatter(j + 1, 1)

                @pl.when(j + 3 < kch)
                def _():
                    gather(j + 3, 1, sem1)

                return carry

            lax.fori_loop(0, npairs, pair_body, 0)
            if kch % 2:
                j = kch - 1
                gwait(j, 0, sem0)
                scatter(j, 0)

            if compute_deg:
                def deg_body(j, carry):
                    pltpu.sync_copy(ones_v, deg_sh.at[dst_v.at[j]], add=True)
                    return carry

                lax.fori_loop(0, kch, deg_body, 0)

            plsc.subcore_barrier()
            wb_acc(r).start()
            if compute_deg:
                wb_deg(r).start()

        wb_acc(2).wait()
        if compute_deg:
            wb_deg(2).wait()

    return body


_BN = 1000


def _tc_combine_body(p_ref, d_ref, w_ref, b_ref, o_ref):
    acc = jnp.broadcast_to(b_ref[...], (_BN, _D))
    for r in range(3):
        agg = p_ref[r, 0] + p_ref[r, 1]
        deg = jnp.maximum(d_ref[r], 1.0)
        acc = acc + jnp.dot(agg / deg, w_ref[r],
                            preferred_element_type=jnp.float32)
    o_ref[...] = acc


def _tc_combine(parts, deg_parts, w, bias_sum):
    return pl.pallas_call(
        _tc_combine_body,
        grid=(_N // _BN,),
        in_specs=[
            pl.BlockSpec((3, _NC, _BN, _D), lambda i: (0, 0, i, 0)),
            pl.BlockSpec((3, _BN, 1), lambda i: (0, i, 0)),
            pl.BlockSpec((3, _D, _D), lambda i: (0, 0, 0)),
            pl.BlockSpec((1, _D), lambda i: (0, 0)),
        ],
        out_specs=pl.BlockSpec((_BN, _D), lambda i: (i, 0)),
        out_shape=jax.ShapeDtypeStruct((_N, _D), jnp.float32),
    )(parts, deg_parts, w, bias_sum)


def _prep_edges(ei, kch):
    e = ei.shape[1]
    t = _NW * kch * _B
    pad = t - e
    src = ei[0].astype(jnp.int32)
    dst = ei[1].astype(jnp.int32)
    if pad:
        ar = jnp.arange(pad, dtype=jnp.int32)
        src = jnp.concatenate([src, (ar * 7919) % _N])
        dst = jnp.concatenate([dst, _N + (ar % (_NPAD - _N))])
    return src.reshape(_NW, kch, _B), dst.reshape(_NW, kch, _B)


def kernel(x, edge_index_r0, edge_index_r1, edge_index_r2,
           W1_r0, b1_r0, W1_r1, b1_r1, W1_r2, b1_r2,
           W2_r0, b2_r0, W2_r1, b2_r1, W2_r2, b2_r2):
    e = edge_index_r0.shape[1]
    kch = -(-e // (_NW * _B))

    prepped = [_prep_edges(ei, kch)
               for ei in (edge_index_r0, edge_index_r1, edge_index_r2)]
    srcs = jnp.stack([p[0] for p in prepped])
    dsts = jnp.stack([p[1] for p in prepped])

    zrow = jnp.zeros((_ZR, _D), jnp.float32)
    ones = jnp.ones((_B,), jnp.float32)
    zdeg = jnp.zeros((_RPT,), jnp.float32)

    w1 = jnp.stack([W1_r0, W1_r1, W1_r2])
    w2 = jnp.stack([W2_r0, W2_r1, W2_r2])
    b1s = (b1_r0 + b1_r1 + b1_r2).reshape(1, _D)
    b2s = (b2_r0 + b2_r1 + b2_r2).reshape(1, _D)

    sc_pass1 = _make_sc_pass(kch, True)
    sc_pass2 = _make_sc_pass(kch, False)

    acc1, deg_flat = sc_pass1(x, srcs, dsts, zrow, ones, zdeg)
    deg = deg_flat.reshape(3, _NC, _NPAD).sum(axis=1).reshape(3, _NPAD, 1)
    h = _tc_combine(acc1, deg, w1, b1s)
    acc2 = sc_pass2(h, srcs, dsts, zrow)
    return _tc_combine(acc2, deg, w2, b2s)


# speedup vs baseline: 1.0037x; 1.0037x over previous
import functools

import jax
import jax.numpy as jnp
from jax import lax
from jax.experimental import pallas as pl
from jax.experimental.pallas import tpu as pltpu
from jax.experimental.pallas import tpu_sc as plsc

_N = 10000
_D = 128
_NPAD = 10240
_NC = 2
_NS = 16
_NW = _NC * _NS
_RPT = _NPAD // _NS
_B = 128
_DEGW = 16
_ZR = 32


@functools.cache
def _make_sc_pass(kch, compute_deg=False):
    mesh = plsc.VectorSubcoreMesh(core_axis_name="c", subcore_axis_name="s")
    out_type = [jax.ShapeDtypeStruct((3, _NC, _NPAD, _D), jnp.float32)]
    scratch = [
        pltpu.VMEM((kch, _B), jnp.int32),
        pltpu.VMEM((kch, _B), jnp.int32),
        pltpu.VMEM((2, _B, _D), jnp.float32),
        pltpu.VMEM((_ZR, _D), jnp.float32),
        pltpu.VMEM_SHARED((_NPAD, _D), jnp.float32),
        pltpu.SemaphoreType.DMA,
        pltpu.SemaphoreType.DMA,
        pltpu.SemaphoreType.DMA,
    ]
    if compute_deg:
        out_type.append(jax.ShapeDtypeStruct((3 * _NC * _NPAD,), jnp.float32))
        scratch += [
            pltpu.VMEM((_B,), jnp.float32),
            pltpu.VMEM((_RPT,), jnp.float32),
            pltpu.VMEM_SHARED((_NPAD,), jnp.float32),
        ]

    @functools.partial(pl.kernel, mesh=mesh,
                       out_type=tuple(out_type) if compute_deg else out_type[0],
                       scratch_types=tuple(scratch))
    def body(*refs):
        if compute_deg:
            (x_hbm, srcs_hbm, dsts_hbm, zrow_hbm, ones_hbm, zdeg_hbm,
             acc_out, deg_out,
             src_v, dst_v, rows_v, zac_v, acc_sh, sem0, sem1, semw,
             ones_v, zdg_v, deg_sh) = refs
        else:
            (x_hbm, srcs_hbm, dsts_hbm, zrow_hbm,
             acc_out,
             src_v, dst_v, rows_v, zac_v, acc_sh, sem0, sem1, semw) = refs
        c = lax.axis_index("c")
        s = lax.axis_index("s")
        wid = s * _NC + c
        base = s * _RPT
        npairs = kch // 2

        pltpu.sync_copy(zrow_hbm, zac_v)
        if compute_deg:
            pltpu.sync_copy(ones_hbm, ones_v)
            pltpu.sync_copy(zdeg_hbm, zdg_v)

        def gather(j, buf, sem):
            pltpu.async_copy(x_hbm.at[src_v.at[j, pl.ds(0, 64)]],
                             rows_v.at[buf, pl.ds(0, 64)], sem)
            pltpu.async_copy(x_hbm.at[src_v.at[j, pl.ds(64, 64)]],
                             rows_v.at[buf, pl.ds(64, 64)], sem)

        def gwait(j, buf, sem):
            pltpu.make_async_copy(x_hbm.at[src_v.at[j, pl.ds(0, 64)]],
                                  rows_v.at[buf, pl.ds(0, 64)], sem).wait()
            pltpu.make_async_copy(x_hbm.at[src_v.at[j, pl.ds(64, 64)]],
                                  rows_v.at[buf, pl.ds(64, 64)], sem).wait()

        def scatter(j, buf):
            pltpu.sync_copy(rows_v.at[buf], acc_sh.at[dst_v.at[j]], add=True)

        def wb_acc(r):
            return pltpu.make_async_copy(acc_sh.at[pl.ds(base, _RPT)],
                                         acc_out.at[r, c, pl.ds(base, _RPT)],
                                         semw)

        def wb_deg(r):
            dst = deg_out.at[pl.ds((r * _NC + c) * _NPAD + base, _RPT)]
            return pltpu.make_async_copy(deg_sh.at[pl.ds(base, _RPT)], dst,
                                         semw)

        for r in range(3):
            pltpu.sync_copy(srcs_hbm.at[r, wid], src_v)
            pltpu.sync_copy(dsts_hbm.at[r, wid], dst_v)
            gather(0, 0, sem0)
            gather(1, 1, sem1)
            if r:
                wb_acc(r - 1).wait()
                if compute_deg:
                    wb_deg(r - 1).wait()
            for k in range(_RPT // _ZR):
                pltpu.sync_copy(zac_v, acc_sh.at[pl.ds(base + k * _ZR, _ZR)])
            if compute_deg:
                pltpu.sync_copy(zdg_v, deg_sh.at[pl.ds(base, _RPT)])
            plsc.subcore_barrier()

            def pair_body(p, carry):
                j = p * 2
                gwait(j, 0, sem0)
                scatter(j, 0)

                @pl.when(j + 2 < kch)
                def _():
                    gather(j + 2, 0, sem0)

                gwait(j + 1, 1, sem1)
                scatter(j + 1, 1)

                @pl.when(j + 3 < kch)
                def _():
                    gather(j + 3, 1, sem1)

                return carry

            lax.fori_loop(0, npairs, pair_body, 0)
            if kch % 2:
                j = kch - 1
                gwait(j, 0, sem0)
                scatter(j, 0)

            if compute_deg:
                def deg_body(j, carry):
                    pltpu.sync_copy(ones_v, deg_sh.at[dst_v.at[j]], add=True)
                    return carry

                lax.fori_loop(0, kch, deg_body, 0)

            plsc.subcore_barrier()
            wb_acc(r).start()
            if compute_deg:
                wb_deg(r).start()

        wb_acc(2).wait()
        if compute_deg:
            wb_deg(2).wait()

    return body


_BN = 1000


def _tc_combine_body(p_ref, d_ref, w_ref, b_ref, o_ref):
    acc = jnp.broadcast_to(b_ref[...], (_BN, _D))
    for r in range(3):
        agg = p_ref[r, 0] + p_ref[r, 1]
        deg = jnp.maximum(d_ref[r], 1.0)
        acc = acc + jnp.dot(agg / deg, w_ref[r],
                            preferred_element_type=jnp.float32)
    o_ref[...] = acc


def _tc_combine(parts, deg_parts, w, bias_sum):
    return pl.pallas_call(
        _tc_combine_body,
        grid=(_N // _BN,),
        in_specs=[
            pl.BlockSpec((3, _NC, _BN, _D), lambda i: (0, 0, i, 0)),
            pl.BlockSpec((3, _BN, 1), lambda i: (0, i, 0)),
            pl.BlockSpec((3, _D, _D), lambda i: (0, 0, 0)),
            pl.BlockSpec((1, _D), lambda i: (0, 0)),
        ],
        out_specs=pl.BlockSpec((_BN, _D), lambda i: (i, 0)),
        out_shape=jax.ShapeDtypeStruct((_N, _D), jnp.float32),
    )(parts, deg_parts, w, bias_sum)


def _prep_edges(ei, kch):
    e = ei.shape[1]
    t = _NW * kch * _B
    pad = t - e
    src = ei[0].astype(jnp.int32)
    dst = ei[1].astype(jnp.int32)
    if pad:
        ar = jnp.arange(pad, dtype=jnp.int32)
        src = jnp.concatenate([src, (ar * 7919) % _N])
        dst = jnp.concatenate([dst, _N + (ar % (_NPAD - _N))])
    return src.reshape(_NW, kch, _B), dst.reshape(_NW, kch, _B)


def kernel(x, edge_index_r0, edge_index_r1, edge_index_r2,
           W1_r0, b1_r0, W1_r1, b1_r1, W1_r2, b1_r2,
           W2_r0, b2_r0, W2_r1, b2_r1, W2_r2, b2_r2):
    e = edge_index_r0.shape[1]
    kch = -(-e // (_NW * _B))

    prepped = [_prep_edges(ei, kch)
               for ei in (edge_index_r0, edge_index_r1, edge_index_r2)]
    srcs = jnp.stack([p[0] for p in prepped])
    dsts = jnp.stack([p[1] for p in prepped])

    zrow = jnp.zeros((_ZR, _D), jnp.float32)
    ones = jnp.ones((_B,), jnp.float32)
    zdeg = jnp.zeros((_RPT,), jnp.float32)

    w1 = jnp.stack([W1_r0, W1_r1, W1_r2])
    w2 = jnp.stack([W2_r0, W2_r1, W2_r2])
    b1s = (b1_r0 + b1_r1 + b1_r2).reshape(1, _D)
    b2s = (b2_r0 + b2_r1 + b2_r2).reshape(1, _D)

    sc_pass1 = _make_sc_pass(kch, True)
    sc_pass2 = _make_sc_pass(kch, False)

    acc1, deg_flat = sc_pass1(x, srcs, dsts, zrow, ones, zdeg)
    deg = deg_flat.reshape(3, _NC, _NPAD).sum(axis=1).reshape(3, _NPAD, 1)
    h = _tc_combine(acc1, deg, w1, b1s)
    acc2 = sc_pass2(h, srcs, dsts, zrow)
    return _tc_combine(acc2, deg, w2, b2s)
